# baseline (speedup 1.0000x reference)
import functools

import jax
import jax.numpy as jnp
from jax import lax
from jax.experimental import pallas as pl
from jax.experimental.pallas import tpu as pltpu
from jax.experimental.pallas import tpu_sc as plsc

CTX = 50
DIM = 64
OUT = 100
KV = CTX * DIM
NC, NS, L = 2, 16, 16
NPT = 4
ACTIVE = OUT // NPT


def _tc_gather_body(sent_s, label_s, wtT_h, ltT_h, out_e, out_l,
                    blocks_v, lblk_v, sem):
    copies = []
    for i in range(CTX):
        base = pl.multiple_of((sent_s[i] // 128) * 128, 128)
        c = pltpu.make_async_copy(
            wtT_h.at[:, pl.ds(base, 128)], blocks_v.at[i], sem)
        c.start()
        copies.append(c)
    lbase = pl.multiple_of((label_s[0] // 128) * 128, 128)
    cl = pltpu.make_async_copy(ltT_h.at[:, pl.ds(lbase, 128)], lblk_v, sem)
    cl.start()
    for i in range(CTX):
        copies[i].wait()
        shift = (128 - sent_s[i] % 128) % 128
        rolled = pltpu.roll(blocks_v[i], shift, axis=1)
        out_e[pl.ds(DIM * i, DIM), :] = rolled[:, 0:1]
    cl.wait()
    lshift = (128 - label_s[0] % 128) % 128
    out_l[...] = pltpu.roll(lblk_v[...], lshift, axis=1)[:, 0:1]


_tc_gather = pl.pallas_call(
    _tc_gather_body,
    out_shape=(
        jax.ShapeDtypeStruct((KV, 1), jnp.float32),
        jax.ShapeDtypeStruct((DIM, 1), jnp.float32),
    ),
    in_specs=[
        pl.BlockSpec(memory_space=pltpu.SMEM),
        pl.BlockSpec(memory_space=pltpu.SMEM),
        pl.BlockSpec(memory_space=pl.ANY),
        pl.BlockSpec(memory_space=pl.ANY),
    ],
    out_specs=(
        pl.BlockSpec(memory_space=pltpu.VMEM),
        pl.BlockSpec(memory_space=pltpu.VMEM),
    ),
    scratch_shapes=[
        pltpu.VMEM((CTX, DIM, 128), jnp.float32),
        pltpu.VMEM((DIM, 128), jnp.float32),
        pltpu.SemaphoreType.DMA,
    ],
    compiler_params=pltpu.CompilerParams(disable_bounds_checks=True),
)


def _lane_sum(x, buf, lane):
    for sh in (8, 4, 2, 1):
        buf[...] = x
        x = x + plsc.load_gather(buf, [lane ^ sh])
    return x


def _sc_dense_body(emb_h, lemb_h, wv_h, bv_h, wl_h, bl_h,
                   out_s_h, out_l_h,
                   emb_v, lemb_v, wv_v, wl_v, bv_v, bl_v,
                   res_s_v, res_l_v, red_v, sem):
    wid = lax.axis_index("s") * NC + lax.axis_index("c")

    @pl.when(wid < ACTIVE)
    def _():
        n0 = wid * NPT
        c_emb = pltpu.async_copy(emb_h, emb_v, sem)
        c_lemb = pltpu.async_copy(lemb_h, lemb_v, sem)
        c_wv = pltpu.async_copy(wv_h.at[pl.ds(n0, NPT)], wv_v, sem)
        c_wl = pltpu.async_copy(wl_h.at[pl.ds(n0, NPT)], wl_v, sem)
        pltpu.sync_copy(bv_h, bv_v)
        pltpu.sync_copy(bl_h, bl_v)
        c_emb.wait()
        c_lemb.wait()
        c_wv.wait()
        c_wl.wait()

        lane = lax.broadcasted_iota(jnp.int32, (L,), 0)

        accs = [jnp.zeros((L,), jnp.float32) for _ in range(NPT)]
        for j in range(KV // L):
            e = emb_v[pl.ds(j * L, L)]
            for r in range(NPT):
                accs[r] = accs[r] + e * wv_v[r, pl.ds(j * L, L)]
        s = [_lane_sum(a, red_v, lane) for a in accs]
        vec = jnp.where(lane == 0, s[0],
              jnp.where(lane == 1, s[1],
              jnp.where(lane == 2, s[2], s[3])))
        bias = plsc.load_gather(bv_v, [jnp.minimum(n0 + lane, OUT - 1)])
        res_s_v[...] = jnp.maximum(vec + bias, 0.0)
        pltpu.sync_copy(res_s_v, out_s_h.at[wid])

        accl = [jnp.zeros((L,), jnp.float32) for _ in range(NPT)]
        for j in range(DIM // L):
            e = lemb_v[pl.ds(j * L, L)]
            for r in range(NPT):
                accl[r] = accl[r] + e * wl_v[r, pl.ds(j * L, L)]
        sl = [_lane_sum(a, red_v, lane) for a in accl]
        vecl = jnp.where(lane == 0, sl[0],
               jnp.where(lane == 1, sl[1],
               jnp.where(lane == 2, sl[2], sl[3])))
        biasl = plsc.load_gather(bl_v, [jnp.minimum(n0 + lane, OUT - 1)])
        res_l_v[...] = jnp.maximum(vecl + biasl, 0.0)
        pltpu.sync_copy(res_l_v, out_l_h.at[wid])


_sc_dense = functools.partial(
    pl.kernel,
    out_type=(
        jax.ShapeDtypeStruct((ACTIVE, L), jnp.float32),
        jax.ShapeDtypeStruct((ACTIVE, L), jnp.float32),
    ),
    mesh=plsc.VectorSubcoreMesh(core_axis_name="c", subcore_axis_name="s",
                                num_cores=NC, num_subcores=NS),
    scratch_types=[
        pltpu.VMEM((KV,), jnp.float32),
        pltpu.VMEM((DIM,), jnp.float32),
        pltpu.VMEM((NPT, KV), jnp.float32),
        pltpu.VMEM((NPT, DIM), jnp.float32),
        pltpu.VMEM((OUT,), jnp.float32),
        pltpu.VMEM((OUT,), jnp.float32),
        pltpu.VMEM((L,), jnp.float32),
        pltpu.VMEM((L,), jnp.float32),
        pltpu.VMEM((L,), jnp.float32),
        pltpu.SemaphoreType.DMA,
    ],
    compiler_params=pltpu.CompilerParams(needs_layout_passes=False,
                                         use_tc_tiling_on_sc=False),
)(_sc_dense_body)


def kernel(sent, label, word_table, label_table, W_vocab, b_vocab, W_label, b_label):
    ecol, lcol = _tc_gather(sent, label, word_table.T, label_table.T)
    out_s, out_l = _sc_dense(ecol.reshape(KV), lcol.reshape(DIM),
                             W_vocab, b_vocab, W_label, b_label)
    sent_out = out_s[:, :NPT].reshape(1, OUT)
    label_out = out_l[:, :NPT].reshape(1, OUT)
    return (sent_out, label_out)


# speedup vs baseline: 4.9772x; 4.9772x over previous
import functools

import jax
import jax.numpy as jnp
from jax.experimental import pallas as pl
from jax.experimental.pallas import tpu as pltpu

CTX = 50
DIM = 64
OUT = 100
KV = CTX * DIM


def _tc_body(sent_s, label_s, wtT_h, ltT_h, wv_h, bv_v, wl_v, bl_v,
             out_s, out_l, blocks_v, lblk_v, ecol_v, lcol_v, wv_v, sem, wsem):
    cw = pltpu.make_async_copy(wv_h, wv_v, wsem)
    cw.start()
    lbase = pl.multiple_of((label_s[0] // 128) * 128, 128)
    cl = pltpu.make_async_copy(ltT_h.at[:, pl.ds(lbase, 128)], lblk_v, sem)
    cl.start()
    copies = []
    for i in range(CTX):
        base = pl.multiple_of((sent_s[i] // 128) * 128, 128)
        c = pltpu.make_async_copy(
            wtT_h.at[:, pl.ds(base, 128)], blocks_v.at[i], sem)
        c.start()
        copies.append(c)
    for i in range(CTX):
        copies[i].wait()
        shift = (128 - sent_s[i] % 128) % 128
        rolled = pltpu.roll(blocks_v[i], shift, axis=1)
        ecol_v[pl.ds(DIM * i, DIM), :] = rolled[:, 0:1]
    cl.wait()
    lshift = (128 - label_s[0] % 128) % 128
    lcol_v[...] = pltpu.roll(lblk_v[...], lshift, axis=1)[:, 0:1]

    cw.wait()
    se = jax.lax.dot_general(ecol_v[...], wv_v[...],
                             (((0,), (1,)), ((), ())),
                             preferred_element_type=jnp.float32)
    out_s[...] = jnp.maximum(se + bv_v[...], 0.0)
    le = jax.lax.dot_general(lcol_v[...], wl_v[...],
                             (((0,), (0,)), ((), ())),
                             preferred_element_type=jnp.float32)
    out_l[...] = jnp.maximum(le + bl_v[...], 0.0)


_tc_call = pl.pallas_call(
    _tc_body,
    out_shape=(
        jax.ShapeDtypeStruct((1, OUT), jnp.float32),
        jax.ShapeDtypeStruct((1, OUT), jnp.float32),
    ),
    in_specs=[
        pl.BlockSpec(memory_space=pltpu.SMEM),
        pl.BlockSpec(memory_space=pltpu.SMEM),
        pl.BlockSpec(memory_space=pl.ANY),
        pl.BlockSpec(memory_space=pl.ANY),
        pl.BlockSpec(memory_space=pl.ANY),
        pl.BlockSpec(memory_space=pltpu.VMEM),
        pl.BlockSpec(memory_space=pltpu.VMEM),
        pl.BlockSpec(memory_space=pltpu.VMEM),
    ],
    out_specs=(
        pl.BlockSpec(memory_space=pltpu.VMEM),
        pl.BlockSpec(memory_space=pltpu.VMEM),
    ),
    scratch_shapes=[
        pltpu.VMEM((CTX, DIM, 128), jnp.float32),
        pltpu.VMEM((DIM, 128), jnp.float32),
        pltpu.VMEM((KV, 1), jnp.float32),
        pltpu.VMEM((DIM, 1), jnp.float32),
        pltpu.VMEM((OUT, KV), jnp.float32),
        pltpu.SemaphoreType.DMA,
        pltpu.SemaphoreType.DMA,
    ],
    compiler_params=pltpu.CompilerParams(disable_bounds_checks=True),
)


def kernel(sent, label, word_table, label_table, W_vocab, b_vocab, W_label, b_label):
    return _tc_call(
        sent, label, word_table.T, label_table.T,
        W_vocab, b_vocab.reshape(1, OUT), W_label.T, b_label.reshape(1, OUT))
